# Initial kernel scaffold; baseline (speedup 1.0000x reference)
import jax
import jax.numpy as jnp
from jax.experimental import pallas as pl


def kernel(x, ray_dir, kp_pos, kp_feat, W1, b1, w_sigma, b_sigma, W_rgb, b_rgb, sample):
    raise NotImplementedError("write your pallas kernel here")


# speedup vs baseline: 20.0210x; 20.0210x over previous
import functools

import jax
import jax.numpy as jnp
from jax.experimental import pallas as pl

K_NN = 8
RADIUS = 0.5
BIG = 3.0e38


def _field_block(pts_ref, dirs_ref, kpt_ref, kpp_ref, kpf_ref,
                 w1a_ref, w1b_ref, b1_ref, ws_ref, bs_ref,
                 wra_ref, wrb_ref, brgb_ref, out_ref):
    blk = pts_ref.shape[0]
    K = kpt_ref.shape[2]

    x = pts_ref[...]
    kpt = kpt_ref[0]

    x2 = jnp.sum(x * x, axis=1, keepdims=True)
    k2 = jnp.sum(kpt * kpt, axis=0, keepdims=True)
    xk = jnp.dot(x, kpt, preferred_element_type=jnp.float32)
    d2 = x2 - 2.0 * xk + k2

    iota_k = jax.lax.broadcasted_iota(jnp.int32, (blk, K), 1)
    kconst = jnp.int32(K)

    wmat = jnp.zeros((blk, K), jnp.float32)
    wsum = jnp.zeros((blk, 1), jnp.float32)
    d0 = None
    d2m = d2
    for j in range(K_NN):
        mval = jnp.min(d2m, axis=1, keepdims=True)
        sel = jnp.where(d2m == mval, iota_k, kconst)
        idxv = jnp.min(sel, axis=1, keepdims=True)
        onehot = sel == idxv
        d2m = jnp.where(onehot, BIG, d2m)
        dj = jnp.maximum(mval, 0.0)
        if j == 0:
            d0 = dj
        wj = 1.0 / (jnp.sqrt(dj) + 1e-8)
        wsum = wsum + wj
        wmat = wmat + jnp.where(onehot, wj, 0.0)

    wmat = wmat * (1.0 / wsum)

    kpf = kpf_ref[0]
    kpp = kpp_ref[0]
    agg_f = jnp.dot(wmat, kpf, preferred_element_type=jnp.float32)
    agg_p = jnp.dot(wmat, kpp, preferred_element_type=jnp.float32)
    rel = x - agg_p

    h = jnp.dot(agg_f, w1a_ref[...], preferred_element_type=jnp.float32)
    h = h + jnp.dot(rel, w1b_ref[...], preferred_element_type=jnp.float32)
    h = jnp.maximum(h + b1_ref[...], 0.0)

    z = jnp.dot(h, ws_ref[...], preferred_element_type=jnp.float32) + bs_ref[...] - 1.0
    sigma = jnp.maximum(z, 0.0) + jnp.log(1.0 + jnp.exp(-jnp.abs(z)))

    dn = dirs_ref[...]
    nrm = jnp.sqrt(jnp.sum(dn * dn, axis=1, keepdims=True))
    dirs = dn / (nrm + 1e-8)

    zr = (jnp.dot(h, wra_ref[...], preferred_element_type=jnp.float32)
          + jnp.dot(dirs, wrb_ref[...], preferred_element_type=jnp.float32)
          + brgb_ref[...])
    rgb = 1.0 / (1.0 + jnp.exp(-zr))

    maskf = jnp.where(d0 < RADIUS * RADIUS, 1.0, 0.0)
    out_ref[...] = jnp.concatenate([sigma, rgb], axis=1) * maskf


def kernel(x, ray_dir, kp_pos, kp_feat, W1, b1, w_sigma, b_sigma, W_rgb, b_rgb, sample):
    B, T, R, S, _ = x.shape
    P = T * R * S
    K = kp_pos.shape[1]
    D = kp_feat.shape[2]
    H = W1.shape[1]

    pts = x.reshape(B * P, 3)
    dirs = jnp.broadcast_to(ray_dir, (B, T, R, S, 3)).reshape(B * P, 3)
    kpt = jnp.transpose(kp_pos, (0, 2, 1))

    W1a = W1[:D]
    W1b = W1[D:]
    Wra = W_rgb[:H]
    Wrb = W_rgb[H:]
    b1r = b1.reshape(1, H)
    bsr = b_sigma.reshape(1, 1)
    brr = b_rgb.reshape(1, 3)

    BLK = 256
    while P % BLK != 0 or (B * P) % BLK != 0:
        BLK //= 2
    nblk = P // BLK

    full = lambda shape: pl.BlockSpec(shape, lambda b, i: (0,) * len(shape))
    per_kp = lambda shape: pl.BlockSpec(shape, lambda b, i: (b, 0, 0))

    out = pl.pallas_call(
        _field_block,
        grid=(B, nblk),
        in_specs=[
            pl.BlockSpec((BLK, 3), lambda b, i: (b * nblk + i, 0)),
            pl.BlockSpec((BLK, 3), lambda b, i: (b * nblk + i, 0)),
            per_kp((1, 3, K)),
            per_kp((1, K, 3)),
            per_kp((1, K, D)),
            full((D, H)), full((3, H)), full((1, H)),
            full((H, 1)), full((1, 1)),
            full((H, 3)), full((3, 3)), full((1, 3)),
        ],
        out_specs=pl.BlockSpec((BLK, 4), lambda b, i: (b * nblk + i, 0)),
        out_shape=jax.ShapeDtypeStruct((B * P, 4), jnp.float32),
    )(pts, dirs, kpt, kp_pos, kp_feat, W1a, W1b, b1r, w_sigma, bsr, Wra, Wrb, brr)

    return out.reshape(B, T, R, S, 4)


# speedup vs baseline: 31.1922x; 1.5580x over previous
import functools

import jax
import jax.numpy as jnp
from jax.experimental import pallas as pl

K_NN = 8
RADIUS = 0.5
BIG = 3.0e38


def _field_block(pts_ref, dirs_ref, kpt_ref, kpp_ref, kpf_ref,
                 w1a_ref, w1b_ref, b1_ref, ws_ref, bs_ref,
                 wra_ref, wrb_ref, brgb_ref, out_ref):
    blk = pts_ref.shape[0]
    K = kpt_ref.shape[2]

    x = pts_ref[...]
    kpt = kpt_ref[0]

    x2 = jnp.sum(x * x, axis=1, keepdims=True)
    k2 = jnp.sum(kpt * kpt, axis=0, keepdims=True)
    xk = jnp.dot(x, kpt, preferred_element_type=jnp.float32)
    d2 = x2 - 2.0 * xk + k2

    wmat = jnp.zeros((blk, K), jnp.float32)
    wsum = jnp.zeros((blk, 1), jnp.float32)
    d0 = None
    d2m = d2
    for j in range(K_NN):
        mval = jnp.min(d2m, axis=1, keepdims=True)
        onehot = d2m == mval
        d2m = jnp.where(onehot, BIG, d2m)
        dj = jnp.maximum(mval, 0.0)
        if j == 0:
            d0 = dj
        wj = 1.0 / (jnp.sqrt(dj) + 1e-8)
        wsum = wsum + wj
        wmat = wmat + jnp.where(onehot, wj, 0.0)

    wmat = wmat * (1.0 / wsum)

    kpf = kpf_ref[0]
    kpp = kpp_ref[0]
    agg_f = jnp.dot(wmat, kpf, preferred_element_type=jnp.float32)
    agg_p = jnp.dot(wmat, kpp, preferred_element_type=jnp.float32)
    rel = x - agg_p

    h = jnp.dot(agg_f, w1a_ref[...], preferred_element_type=jnp.float32)
    h = h + jnp.dot(rel, w1b_ref[...], preferred_element_type=jnp.float32)
    h = jnp.maximum(h + b1_ref[...], 0.0)

    z = jnp.dot(h, ws_ref[...], preferred_element_type=jnp.float32) + bs_ref[...] - 1.0
    sigma = jnp.maximum(z, 0.0) + jnp.log(1.0 + jnp.exp(-jnp.abs(z)))

    dn = dirs_ref[...]
    nrm = jnp.sqrt(jnp.sum(dn * dn, axis=1, keepdims=True))
    dirs = dn / (nrm + 1e-8)

    zr = (jnp.dot(h, wra_ref[...], preferred_element_type=jnp.float32)
          + jnp.dot(dirs, wrb_ref[...], preferred_element_type=jnp.float32)
          + brgb_ref[...])
    rgb = 1.0 / (1.0 + jnp.exp(-zr))

    maskf = jnp.where(d0 < RADIUS * RADIUS, 1.0, 0.0)
    out_ref[...] = jnp.concatenate([sigma, rgb], axis=1) * maskf


def kernel(x, ray_dir, kp_pos, kp_feat, W1, b1, w_sigma, b_sigma, W_rgb, b_rgb, sample):
    B, T, R, S, _ = x.shape
    P = T * R * S
    K = kp_pos.shape[1]
    D = kp_feat.shape[2]
    H = W1.shape[1]

    pts = x.reshape(B * P, 3)
    dirs = jnp.broadcast_to(ray_dir, (B, T, R, S, 3)).reshape(B * P, 3)
    kpt = jnp.transpose(kp_pos, (0, 2, 1))

    W1a = W1[:D]
    W1b = W1[D:]
    Wra = W_rgb[:H]
    Wrb = W_rgb[H:]
    b1r = b1.reshape(1, H)
    bsr = b_sigma.reshape(1, 1)
    brr = b_rgb.reshape(1, 3)

    BLK = 256
    while P % BLK != 0 or (B * P) % BLK != 0:
        BLK //= 2
    nblk = P // BLK

    full = lambda shape: pl.BlockSpec(shape, lambda b, i: (0,) * len(shape))
    per_kp = lambda shape: pl.BlockSpec(shape, lambda b, i: (b, 0, 0))

    out = pl.pallas_call(
        _field_block,
        grid=(B, nblk),
        in_specs=[
            pl.BlockSpec((BLK, 3), lambda b, i: (b * nblk + i, 0)),
            pl.BlockSpec((BLK, 3), lambda b, i: (b * nblk + i, 0)),
            per_kp((1, 3, K)),
            per_kp((1, K, 3)),
            per_kp((1, K, D)),
            full((D, H)), full((3, H)), full((1, H)),
            full((H, 1)), full((1, 1)),
            full((H, 3)), full((3, 3)), full((1, 3)),
        ],
        out_specs=pl.BlockSpec((BLK, 4), lambda b, i: (b * nblk + i, 0)),
        out_shape=jax.ShapeDtypeStruct((B * P, 4), jnp.float32),
    )(pts, dirs, kpt, kp_pos, kp_feat, W1a, W1b, b1r, w_sigma, bsr, Wra, Wrb, brr)

    return out.reshape(B, T, R, S, 4)
